# baseline (speedup 1.0000x reference)
import functools

import jax
import jax.numpy as jnp
from jax import lax
from jax.experimental import pallas as pl
from jax.experimental.pallas import tpu as pltpu
from jax.experimental.pallas import tpu_sc as plsc

_INFO = plsc.get_sparse_core_info()
_NC = _INFO.num_cores
_NS = _INFO.num_subcores
_NW = _NC * _NS
_L = _INFO.num_lanes


@jax.jit
def _sc_gather_t(tok_t, pe_weight):
    S, B0 = tok_t.shape
    V, D = pe_weight.shape
    DT, DI = D // 8, 8
    BT, BI = B0 // 128, 128
    BW = BT // _NW
    TOK_W = BW * BI
    SB = 20
    NG = S // SB

    mesh = plsc.VectorSubcoreMesh(core_axis_name="c", subcore_axis_name="s")

    @functools.partial(
        pl.kernel,
        mesh=mesh,
        out_type=jax.ShapeDtypeStruct((S, DT, BT, DI, BI), jnp.float32),
        scratch_types=[
            pltpu.VMEM((SB, TOK_W), jnp.int32),
            pltpu.VMEM((2, TOK_W, D), jnp.float32),
            pltpu.VMEM((2, D, TOK_W + 1), jnp.float32),
            pltpu.SemaphoreType.DMA,
            pltpu.SemaphoreType.DMA,
            pltpu.SemaphoreType.DMA,
            pltpu.SemaphoreType.DMA,
        ],
        compiler_params=pltpu.CompilerParams(
            use_tc_tiling_on_sc=False, needs_layout_passes=False
        ),
    )
    def k(idx_hbm, table_hbm, out_hbm, idx_v, rows_v, trans_v, sg0, sg1,
          so0, so1):
        wid = lax.axis_index("s") * _NC + lax.axis_index("c")
        b0 = wid * TOK_W
        iota = lax.iota(jnp.int32, _L)
        nj = TOK_W // _L
        sgs = (sg0, sg1)
        sos = (so0, so1)

        def gather(sl, p):
            return pltpu.make_async_copy(
                table_hbm.at[idx_v.at[sl]], rows_v.at[p], sgs[p]
            )

        def out_copies(s, p):
            return [
                pltpu.make_async_copy(
                    trans_v.at[p, pl.ds(dt * DI, DI), pl.ds(btl * BI, BI)],
                    out_hbm.at[s, dt, wid * BW + btl],
                    sos[p],
                )
                for dt in range(DT)
                for btl in range(BW)
            ]

        def transpose(p):
            rows = rows_v.at[p]
            tr = trans_v.at[p]
            dvs = [d0 + iota for d0 in range(0, D, _L)]

            @plsc.parallel_loop(0, TOK_W, unroll=4)
            def _(t):
                tok = jnp.full((_L,), t, jnp.int32)
                for h, dv in enumerate(dvs):
                    vec = rows[t, pl.ds(h * _L, _L)]
                    plsc.store_scatter(tr, [dv, tok], vec)

        def step(gi, sl, p):
            s = gi * SB + sl
            gather(sl, p).wait()

            @pl.when(s >= 2)
            def _():
                for c in out_copies(jnp.maximum(s - 2, 0), p):
                    c.wait()

            transpose(p)
            gather(jnp.minimum(sl + 2, SB - 1), p).start()
            for c in out_copies(s, p):
                c.start()

        def outer(gi, carry):
            pltpu.sync_copy(
                idx_hbm.at[pl.ds(gi * SB, SB), pl.ds(b0, TOK_W)], idx_v
            )
            gather(0, 0).start()
            gather(1, 1).start()

            def inner(si, c2):
                step(gi, 2 * si, 0)
                step(gi, 2 * si + 1, 1)
                return c2

            lax.fori_loop(0, SB // 2, inner, 0)
            gather(SB - 1, 0).wait()
            gather(SB - 1, 1).wait()
            return carry

        lax.fori_loop(0, NG, outer, 0)
        for p in range(2):
            for c in out_copies(S - 2 + p, p):
                c.wait()

    return k(tok_t, pe_weight)


def kernel(tok, pe_weight):
    B0, S = tok.shape
    V, D = pe_weight.shape
    out_lin = _sc_gather_t(tok.T, pe_weight)
    return out_lin.transpose(2, 4, 0, 1, 3).reshape(B0, S, D)


# speedup vs baseline: 1.0527x; 1.0527x over previous
import functools

import jax
import jax.numpy as jnp
from jax import lax
from jax.experimental import pallas as pl
from jax.experimental.pallas import tpu as pltpu
from jax.experimental.pallas import tpu_sc as plsc

_INFO = plsc.get_sparse_core_info()
_NC = _INFO.num_cores
_NS = _INFO.num_subcores
_NW = _NC * _NS
_L = _INFO.num_lanes


@jax.jit
def _sc_gather_t(tok_t, pe_weight):
    S, B0 = tok_t.shape
    V, D = pe_weight.shape
    DT, DI = D // 8, 8
    BT, BI = B0 // 128, 128
    BW = BT // _NW
    TOK_W = BW * BI
    SB = 50
    NG = S // SB

    mesh = plsc.VectorSubcoreMesh(core_axis_name="c", subcore_axis_name="s")

    @functools.partial(
        pl.kernel,
        mesh=mesh,
        out_type=jax.ShapeDtypeStruct((S, DT, BT, DI, BI), jnp.float32),
        scratch_types=[
            pltpu.VMEM((SB, TOK_W), jnp.int32),
            pltpu.VMEM((2, TOK_W, D), jnp.float32),
            pltpu.VMEM((2, D, TOK_W + 1), jnp.float32),
            pltpu.SemaphoreType.DMA,
            pltpu.SemaphoreType.DMA,
            pltpu.SemaphoreType.DMA,
            pltpu.SemaphoreType.DMA,
        ],
        compiler_params=pltpu.CompilerParams(
            use_tc_tiling_on_sc=False, needs_layout_passes=False
        ),
    )
    def k(idx_hbm, table_hbm, out_hbm, idx_v, rows_v, trans_v, sg0, sg1,
          so0, so1):
        wid = lax.axis_index("s") * _NC + lax.axis_index("c")
        b0 = wid * TOK_W
        iota = lax.iota(jnp.int32, _L)
        nj = TOK_W // _L
        sgs = (sg0, sg1)
        sos = (so0, so1)

        def gather(sl, p):
            return pltpu.make_async_copy(
                table_hbm.at[idx_v.at[sl]], rows_v.at[p], sgs[p]
            )

        def out_copies(s, p):
            return [
                pltpu.make_async_copy(
                    trans_v.at[p, pl.ds(dt * DI, DI), pl.ds(btl * BI, BI)],
                    out_hbm.at[s, dt, wid * BW + btl],
                    sos[p],
                )
                for dt in range(DT)
                for btl in range(BW)
            ]

        def transpose(p):
            rows = rows_v.at[p]
            tr = trans_v.at[p]
            dvs = [d0 + iota for d0 in range(0, D, _L)]

            @plsc.parallel_loop(0, TOK_W, unroll=4)
            def _(t):
                tok = jnp.full((_L,), t, jnp.int32)
                for h, dv in enumerate(dvs):
                    vec = rows[t, pl.ds(h * _L, _L)]
                    plsc.store_scatter(tr, [dv, tok], vec)

        def step(gi, sl, p):
            s = gi * SB + sl
            gather(sl, p).wait()

            @pl.when(s >= 2)
            def _():
                for c in out_copies(jnp.maximum(s - 2, 0), p):
                    c.wait()

            transpose(p)
            gather(jnp.minimum(sl + 2, SB - 1), p).start()
            for c in out_copies(s, p):
                c.start()

        def outer(gi, carry):
            pltpu.sync_copy(
                idx_hbm.at[pl.ds(gi * SB, SB), pl.ds(b0, TOK_W)], idx_v
            )
            gather(0, 0).start()
            gather(1, 1).start()

            def inner(si, c2):
                step(gi, 2 * si, 0)
                step(gi, 2 * si + 1, 1)
                return c2

            lax.fori_loop(0, SB // 2, inner, 0)
            gather(SB - 1, 0).wait()
            gather(SB - 1, 1).wait()
            return carry

        lax.fori_loop(0, NG, outer, 0)
        for p in range(2):
            for c in out_copies(S - 2 + p, p):
                c.wait()

    return k(tok_t, pe_weight)


def kernel(tok, pe_weight):
    B0, S = tok.shape
    V, D = pe_weight.shape
    out_lin = _sc_gather_t(tok.T, pe_weight)
    return out_lin.transpose(2, 4, 0, 1, 3).reshape(B0, S, D)


# speedup vs baseline: 1.0693x; 1.0157x over previous
import functools

import jax
import jax.numpy as jnp
from jax import lax
from jax.experimental import pallas as pl
from jax.experimental.pallas import tpu as pltpu
from jax.experimental.pallas import tpu_sc as plsc

_INFO = plsc.get_sparse_core_info()
_NC = _INFO.num_cores
_NS = _INFO.num_subcores
_NW = _NC * _NS
_L = _INFO.num_lanes


@jax.jit
def _sc_gather_t(tok_t, pe_weight):
    S, B0 = tok_t.shape
    V, D = pe_weight.shape
    DT, DI = D // 8, 8
    BT, BI = B0 // 128, 128
    BW = BT // _NW
    TOK_W = BW * BI
    SB = 100
    NG = S // SB

    mesh = plsc.VectorSubcoreMesh(core_axis_name="c", subcore_axis_name="s")

    @functools.partial(
        pl.kernel,
        mesh=mesh,
        out_type=jax.ShapeDtypeStruct((S, DT, BT, DI, BI), jnp.float32),
        scratch_types=[
            pltpu.VMEM((SB, TOK_W), jnp.int32),
            pltpu.VMEM((2, TOK_W, D), jnp.float32),
            pltpu.VMEM((2, D, TOK_W + 1), jnp.float32),
            pltpu.SemaphoreType.DMA,
            pltpu.SemaphoreType.DMA,
            pltpu.SemaphoreType.DMA,
            pltpu.SemaphoreType.DMA,
        ],
        compiler_params=pltpu.CompilerParams(
            use_tc_tiling_on_sc=False, needs_layout_passes=False
        ),
    )
    def k(idx_hbm, table_hbm, out_hbm, idx_v, rows_v, trans_v, sg0, sg1,
          so0, so1):
        wid = lax.axis_index("s") * _NC + lax.axis_index("c")
        b0 = wid * TOK_W
        iota = lax.iota(jnp.int32, _L)
        nj = TOK_W // _L
        sgs = (sg0, sg1)
        sos = (so0, so1)

        def gather(sl, p):
            return pltpu.make_async_copy(
                table_hbm.at[idx_v.at[sl]], rows_v.at[p], sgs[p]
            )

        def out_copies(s, p):
            return [
                pltpu.make_async_copy(
                    trans_v.at[p, pl.ds(dt * DI, DI), pl.ds(btl * BI, BI)],
                    out_hbm.at[s, dt, wid * BW + btl],
                    sos[p],
                )
                for dt in range(DT)
                for btl in range(BW)
            ]

        def transpose(p):
            rows = rows_v.at[p]
            tr = trans_v.at[p]
            dvs = [d0 + iota for d0 in range(0, D, _L)]

            @plsc.parallel_loop(0, TOK_W, unroll=4)
            def _(t):
                tok = jnp.full((_L,), t, jnp.int32)
                for h, dv in enumerate(dvs):
                    vec = rows[t, pl.ds(h * _L, _L)]
                    plsc.store_scatter(tr, [dv, tok], vec)

        def step(gi, sl, p):
            s = gi * SB + sl
            gather(sl, p).wait()

            @pl.when(s >= 2)
            def _():
                for c in out_copies(jnp.maximum(s - 2, 0), p):
                    c.wait()

            transpose(p)
            gather(jnp.minimum(sl + 2, SB - 1), p).start()
            for c in out_copies(s, p):
                c.start()

        def outer(gi, carry):
            pltpu.sync_copy(
                idx_hbm.at[pl.ds(gi * SB, SB), pl.ds(b0, TOK_W)], idx_v
            )
            gather(0, 0).start()
            gather(1, 1).start()

            def inner(si, c2):
                step(gi, 2 * si, 0)
                step(gi, 2 * si + 1, 1)
                return c2

            lax.fori_loop(0, SB // 2, inner, 0)
            gather(SB - 1, 0).wait()
            gather(SB - 1, 1).wait()
            return carry

        lax.fori_loop(0, NG, outer, 0)
        for p in range(2):
            for c in out_copies(S - 2 + p, p):
                c.wait()

    return k(tok_t, pe_weight)


def kernel(tok, pe_weight):
    B0, S = tok.shape
    V, D = pe_weight.shape
    out_lin = _sc_gather_t(tok.T, pe_weight)
    return out_lin.transpose(2, 4, 0, 1, 3).reshape(B0, S, D)


# speedup vs baseline: 1.0731x; 1.0035x over previous
import functools

import jax
import jax.numpy as jnp
from jax import lax
from jax.experimental import pallas as pl
from jax.experimental.pallas import tpu as pltpu
from jax.experimental.pallas import tpu_sc as plsc

_INFO = plsc.get_sparse_core_info()
_NC = _INFO.num_cores
_NS = _INFO.num_subcores
_NW = _NC * _NS
_L = _INFO.num_lanes


@jax.jit
def _sc_gather_t(tok_t, pe_weight):
    S, B0 = tok_t.shape
    V, D = pe_weight.shape
    DT, DI = D // 8, 8
    BT, BI = B0 // 128, 128
    BW = BT // _NW
    TOK_W = BW * BI
    SB = 100
    NG = S // SB

    mesh = plsc.VectorSubcoreMesh(core_axis_name="c", subcore_axis_name="s")

    @functools.partial(
        pl.kernel,
        mesh=mesh,
        out_type=jax.ShapeDtypeStruct((S, DT, BT, DI, BI), jnp.float32),
        scratch_types=[
            pltpu.VMEM((SB, TOK_W), jnp.int32),
            pltpu.VMEM((2, TOK_W, D), jnp.float32),
            pltpu.VMEM((2, D, TOK_W + 1), jnp.float32),
            pltpu.SemaphoreType.DMA,
            pltpu.SemaphoreType.DMA,
            pltpu.SemaphoreType.DMA,
            pltpu.SemaphoreType.DMA,
        ],
        compiler_params=pltpu.CompilerParams(
            use_tc_tiling_on_sc=False, needs_layout_passes=False
        ),
    )
    def k(idx_hbm, table_hbm, out_hbm, idx_v, rows_v, trans_v, sg0, sg1,
          so0, so1):
        wid = lax.axis_index("s") * _NC + lax.axis_index("c")
        b0 = wid * TOK_W
        iota = lax.iota(jnp.int32, _L)
        sgs = (sg0, sg1)
        sos = (so0, so1)

        def gather(sl, p):
            return pltpu.make_async_copy(
                table_hbm.at[idx_v.at[sl]], rows_v.at[p], sgs[p]
            )

        def out_copies(s, p):
            return [
                pltpu.make_async_copy(
                    trans_v.at[p, pl.ds(dt * DI, DI), pl.ds(btl * BI, BI)],
                    out_hbm.at[s, dt, wid * BW + btl],
                    sos[p],
                )
                for dt in range(DT)
                for btl in range(BW)
            ]

        def transpose(p):
            rows = rows_v.at[p]
            tr = trans_v.at[p]
            dvs = [d0 + iota for d0 in range(0, D, _L)]

            @plsc.parallel_loop(0, TOK_W, unroll=4)
            def _(t):
                tok = jnp.full((_L,), t, jnp.int32)
                for h, dv in enumerate(dvs):
                    vec = rows[t, pl.ds(h * _L, _L)]
                    plsc.store_scatter(tr, [dv, tok], vec)

        def step(gi, sl, p):
            s = gi * SB + sl
            gather(sl, p).wait()

            @pl.when(s >= 2)
            def _():
                for c in out_copies(jnp.maximum(s - 2, 0), p):
                    c.wait()

            transpose(p)
            gather(jnp.minimum(sl + 2, SB - 1), p).start()
            for c in out_copies(s, p):
                c.start()

        def outer(gi, carry):
            pltpu.sync_copy(
                idx_hbm.at[pl.ds(gi * SB, SB), pl.ds(b0, TOK_W)], idx_v
            )
            gather(0, 0).start()
            gather(1, 1).start()

            def inner(si, c2):
                step(gi, 2 * si, 0)
                step(gi, 2 * si + 1, 1)
                return c2

            lax.fori_loop(0, SB // 2, inner, 0)
            gather(SB - 1, 0).wait()
            gather(SB - 1, 1).wait()
            return carry

        lax.fori_loop(0, NG, outer, 0)
        for p in range(2):
            for c in out_copies(S - 2 + p, p):
                c.wait()

    return k(tok_t, pe_weight)


def kernel(tok, pe_weight):
    B0, S = tok.shape
    V, D = pe_weight.shape
    out_lin = _sc_gather_t(tok.T, pe_weight)
    return out_lin.transpose(2, 4, 0, 1, 3).reshape(B0, S, D)
